# Initial kernel scaffold; baseline (speedup 1.0000x reference)
import functools

import jax
import jax.numpy as jnp
from jax import lax
from jax.experimental import pallas as pl
from jax.experimental.pallas import tpu as pltpu
from jax.experimental.pallas import tpu_sc as plsc

_K = 16


def _lrelu(v, s):
    return jnp.where(v >= 0, v, s * v)


def _knn_body(q_ref, s_ref, o_ref, *, n, st, k):
    qb = q_ref.shape[0]
    qx = q_ref[:, 0:1]
    qy = q_ref[:, 1:2]
    qz = q_ref[:, 2:3]
    it = lax.broadcasted_iota(jnp.int32, (qb, st), 1)
    slot = lax.broadcasted_iota(jnp.int32, (1, k), 1)

    def tile_step(t, carry):
        rv, ri = carry
        sx = s_ref[0:1, pl.ds(t * st, st)]
        sy = s_ref[1:2, pl.ds(t * st, st)]
        sz = s_ref[2:3, pl.ds(t * st, st)]
        d = (qx - sx) ** 2 + (qy - sy) ** 2 + (qz - sz) ** 2

        def round_step(_, carry2):
            d, rv, ri = carry2
            m = jnp.min(d, axis=1, keepdims=True)
            a = jnp.where(d == m, it, n)
            i = jnp.min(a, axis=1, keepdims=True)
            d = jnp.where(it == i, jnp.inf, d)
            w = jnp.max(rv, axis=1, keepdims=True)
            ws = jnp.where(rv == w, slot, k)
            s0 = jnp.min(ws, axis=1, keepdims=True)
            enter = m < w
            sel = (slot == s0) & enter
            rv = jnp.where(sel, m, rv)
            ri = jnp.where(sel, i + t * st, ri)
            return d, rv, ri

        _, rv, ri = lax.fori_loop(0, k, round_step, (d, rv, ri))
        return rv, ri

    rv0 = jnp.full((qb, k), jnp.inf, jnp.float32)
    ri0 = jnp.zeros((qb, k), jnp.int32)
    _, ri = lax.fori_loop(0, n // st, tile_step, (rv0, ri0))
    o_ref[...] = ri


def _knn(q, s_t, k):
    nq = q.shape[0]
    n = s_t.shape[1]
    qb = min(128, nq)
    st = min(2048, n)
    return pl.pallas_call(
        functools.partial(_knn_body, n=n, st=st, k=k),
        grid=(nq // qb,),
        in_specs=[
            pl.BlockSpec((qb, 3), lambda i: (i, 0)),
            pl.BlockSpec((3, n), lambda i: (0, 0)),
        ],
        out_specs=pl.BlockSpec((qb, k), lambda i: (i, 0)),
        out_shape=jax.ShapeDtypeStruct((nq, k), jnp.int32),
    )(q, s_t)


def _nn1_body(q_ref, s_ref, o_ref, *, n, st):
    qb = q_ref.shape[0]
    qx = q_ref[:, 0:1]
    qy = q_ref[:, 1:2]
    qz = q_ref[:, 2:3]
    it = lax.broadcasted_iota(jnp.int32, (qb, st), 1)

    def tile_step(t, carry):
        bv, bi = carry
        sx = s_ref[0:1, pl.ds(t * st, st)]
        sy = s_ref[1:2, pl.ds(t * st, st)]
        sz = s_ref[2:3, pl.ds(t * st, st)]
        d = (qx - sx) ** 2 + (qy - sy) ** 2 + (qz - sz) ** 2
        m = jnp.min(d, axis=1, keepdims=True)
        a = jnp.where(d == m, it, n)
        i = jnp.min(a, axis=1, keepdims=True) + t * st
        better = m < bv
        bv = jnp.where(better, m, bv)
        bi = jnp.where(better, i, bi)
        return bv, bi

    bv0 = jnp.full((qb, 1), jnp.inf, jnp.float32)
    bi0 = jnp.zeros((qb, 1), jnp.int32)
    _, bi = lax.fori_loop(0, n // st, tile_step, (bv0, bi0))
    o_ref[...] = bi


def _nn1(q, s_t):
    nq = q.shape[0]
    n = s_t.shape[1]
    qb = min(128, nq)
    st = min(2048, n)
    out = pl.pallas_call(
        functools.partial(_nn1_body, n=n, st=st),
        grid=(nq // qb,),
        in_specs=[
            pl.BlockSpec((qb, 3), lambda i: (i, 0)),
            pl.BlockSpec((3, n), lambda i: (0, 0)),
        ],
        out_specs=pl.BlockSpec((qb, 1), lambda i: (i, 0)),
        out_shape=jax.ShapeDtypeStruct((nq, 1), jnp.int32),
    )(q, s_t)
    return out[:, 0]


def _lin_body(x_ref, w_ref, b_ref, o_ref, *, act, slope):
    y = lax.dot_general(x_ref[...], w_ref[...], (((1,), (1,)), ((), ())),
                        preferred_element_type=jnp.float32)
    y = y + b_ref[...]
    if act == "lrelu":
        y = _lrelu(y, slope)
    elif act == "relu":
        y = jnp.maximum(y, 0.0)
    o_ref[...] = y


def _lin_res_body(x_ref, w_ref, b_ref, r_ref, o_ref, *, slope, slope2):
    y = lax.dot_general(x_ref[...], w_ref[...], (((1,), (1,)), ((), ())),
                        preferred_element_type=jnp.float32)
    y = _lrelu(y + b_ref[...], slope)
    o_ref[...] = _lrelu(y + r_ref[...], slope2)


def _linear(x, w, b, act="none", slope=0.2):
    n, din = x.shape
    dout = w.shape[0]
    nb = min(512, n)
    return pl.pallas_call(
        functools.partial(_lin_body, act=act, slope=slope),
        grid=(n // nb,),
        in_specs=[
            pl.BlockSpec((nb, din), lambda i: (i, 0)),
            pl.BlockSpec((dout, din), lambda i: (0, 0)),
            pl.BlockSpec((1, dout), lambda i: (0, 0)),
        ],
        out_specs=pl.BlockSpec((nb, dout), lambda i: (i, 0)),
        out_shape=jax.ShapeDtypeStruct((n, dout), jnp.float32),
    )(x, w, b.reshape(1, -1))


def _linear_res(x, w, b, res, slope=0.2, slope2=0.01):
    n, din = x.shape
    dout = w.shape[0]
    nb = min(512, n)
    return pl.pallas_call(
        functools.partial(_lin_res_body, slope=slope, slope2=slope2),
        grid=(n // nb,),
        in_specs=[
            pl.BlockSpec((nb, din), lambda i: (i, 0)),
            pl.BlockSpec((dout, din), lambda i: (0, 0)),
            pl.BlockSpec((1, dout), lambda i: (0, 0)),
            pl.BlockSpec((nb, dout), lambda i: (i, 0)),
        ],
        out_specs=pl.BlockSpec((nb, dout), lambda i: (i, 0)),
        out_shape=jax.ShapeDtypeStruct((n, dout), jnp.float32),
    )(x, w, b.reshape(1, -1), res)


def _lfa_body(xj_ref, pi_ref, pj_ref, ew_ref, eb_ref, aw_ref, ab_ref, o_ref,
              *, k, qb):
    pi = pi_ref[...]
    pj = pj_ref[...]
    dist = pj - pi
    eu = jnp.sum(jnp.abs(dist), axis=1, keepdims=True)
    rel = jnp.concatenate([pi, pj, dist, eu], axis=1)
    lse = lax.dot_general(rel, ew_ref[...], (((1,), (1,)), ((), ())),
                          preferred_element_type=jnp.float32) + eb_ref[...]
    out1 = jnp.concatenate([xj_ref[...], lse], axis=1)
    logits = lax.dot_general(out1, aw_ref[...], (((1,), (1,)), ((), ())),
                             preferred_element_type=jnp.float32) + ab_ref[...]
    z = logits - jnp.max(logits, axis=1, keepdims=True)
    e = jnp.exp(z)
    att = e / jnp.sum(e, axis=1, keepdims=True)
    msg = att * out1
    eb = qb * k
    rows = lax.broadcasted_iota(jnp.int32, (qb, eb), 0)
    cols = lax.broadcasted_iota(jnp.int32, (qb, eb), 1)
    sel = jnp.where(cols // k == rows, 1.0, 0.0)
    o_ref[...] = lax.dot_general(sel, msg, (((1,), (0,)), ((), ())),
                                 preferred_element_type=jnp.float32)


def _lfa(xj, pos_i, pos_j, ew, ebias, aw, abias, k):
    e, c1 = xj.shape
    nq = e // k
    d = aw.shape[0]
    qb = min(64 if d >= 256 else 128, nq)
    eb = qb * k
    return pl.pallas_call(
        functools.partial(_lfa_body, k=k, qb=qb),
        grid=(nq // qb,),
        in_specs=[
            pl.BlockSpec((eb, c1), lambda i: (i, 0)),
            pl.BlockSpec((eb, 3), lambda i: (i, 0)),
            pl.BlockSpec((eb, 3), lambda i: (i, 0)),
            pl.BlockSpec(ew.shape, lambda i: (0, 0)),
            pl.BlockSpec((1, ew.shape[0]), lambda i: (0, 0)),
            pl.BlockSpec(aw.shape, lambda i: (0, 0)),
            pl.BlockSpec((1, d), lambda i: (0, 0)),
        ],
        out_specs=pl.BlockSpec((qb, d), lambda i: (i, 0)),
        out_shape=jax.ShapeDtypeStruct((nq, d), jnp.float32),
    )(xj, pos_i, pos_j, ew, ebias.reshape(1, -1), aw, abias.reshape(1, -1))


def _sc_gather(table, idx):
    v, dd = table.shape
    b = idx.shape[0]
    info = plsc.get_sparse_core_info()
    nw = info.num_cores * info.num_subcores
    bpw = b // nw
    ch = min(128, bpw)
    nch = bpw // ch
    mesh = plsc.VectorSubcoreMesh(core_axis_name="c", subcore_axis_name="s")

    @functools.partial(
        pl.kernel, mesh=mesh,
        out_type=jax.ShapeDtypeStruct((b, dd), jnp.float32),
        scratch_types=[
            pltpu.VMEM((bpw,), jnp.int32),
            pltpu.VMEM((bpw, dd), jnp.float32),
            pltpu.SemaphoreType.DMA,
        ],
    )
    def gk(table_hbm, idx_hbm, out_hbm, idx_v, rows_v, sem):
        wid = lax.axis_index("s") * info.num_cores + lax.axis_index("c")
        base = wid * bpw
        pltpu.sync_copy(idx_hbm.at[pl.ds(base, bpw)], idx_v)

        def body(j, carry):
            off = pl.multiple_of(j * ch, 8)
            pltpu.async_copy(table_hbm.at[idx_v.at[pl.ds(off, ch)]],
                             rows_v.at[pl.ds(off, ch)], sem).wait()
            return carry

        lax.fori_loop(0, nch, body, 0)
        pltpu.sync_copy(rows_v, out_hbm.at[pl.ds(base, bpw)])

    return gk(table, idx)


def _pad_cols(a, mult):
    n, c = a.shape
    pad = (-c) % mult
    if pad:
        a = jnp.concatenate([a, jnp.zeros((n, pad), a.dtype)], axis=1)
    return a


def _gather_rows(table, idx, want_cols):
    tp = _pad_cols(table, 16)
    return _sc_gather(tp, idx)[:, :want_cols]


def _block_fwd(params, pfx, x, pos, dec, k):
    n = x.shape[0]
    nq = n // dec
    q = pos[::dec]
    nbrs = _knn(q, pos.T, k)
    col = nbrs.reshape(-1)
    sc = _linear(x, params[pfx + "_sc_W"], params[pfx + "_sc_b"], "lrelu", 0.2)
    hm = _linear(x, params[pfx + "_m1_W"], params[pfx + "_m1_b"], "lrelu", 0.2)
    pos_j = _gather_rows(pos, col, 3)
    pos_i = jnp.repeat(q, k, axis=0)
    xj = _gather_rows(hm, col, hm.shape[1])
    h1q = _lfa(xj, pos_i, pos_j,
               params[pfx + "_l1_e_W"], params[pfx + "_l1_e_b"],
               params[pfx + "_l1_a_W"], params[pfx + "_l1_a_b"], k)
    d1 = h1q.shape[1]
    h1full = jnp.concatenate(
        [h1q, jnp.zeros((n - nq, d1), jnp.float32)], axis=0)
    xj2 = _gather_rows(h1full, col, d1)
    h2q = _lfa(xj2, pos_i, pos_j,
               params[pfx + "_l2_e_W"], params[pfx + "_l2_e_b"],
               params[pfx + "_l2_a_W"], params[pfx + "_l2_a_b"], k)
    d2 = h2q.shape[1]
    h2full = jnp.concatenate(
        [h2q, jnp.zeros((n - nq, d2), jnp.float32)], axis=0)
    h4 = _linear_res(h2full, params[pfx + "_m2_W"], params[pfx + "_m2_b"], sc)
    return h4[::dec], q


def _fp_fwd(params, pfx, xh, pos, pos_skip, x_skip):
    nn = _nn1(pos_skip, pos.T)
    xi = _gather_rows(xh, nn, xh.shape[1])
    if x_skip is not None:
        xi = jnp.concatenate([xi, x_skip], axis=1)
    return _linear(xi, params[pfx + "_W"], params[pfx + "_b"], "none")


def _head_body(x_ref, w1_ref, b1_ref, w2_ref, b2_ref, w3_ref, b3_ref, o_ref):
    h = lax.dot_general(x_ref[...], w1_ref[...], (((1,), (1,)), ((), ())),
                        preferred_element_type=jnp.float32) + b1_ref[...]
    h = jnp.maximum(h, 0.0)
    h = lax.dot_general(h, w2_ref[...], (((1,), (1,)), ((), ())),
                        preferred_element_type=jnp.float32) + b2_ref[...]
    h = lax.dot_general(h, w3_ref[...], (((1,), (1,)), ((), ())),
                        preferred_element_type=jnp.float32) + b3_ref[...]
    o_ref[...] = h


def _head(x, w1, b1, w2, b2, w3, b3):
    n = x.shape[0]
    nb = min(512, n)
    d3 = w3.shape[0]
    return pl.pallas_call(
        _head_body,
        grid=(n // nb,),
        in_specs=[
            pl.BlockSpec((nb, x.shape[1]), lambda i: (i, 0)),
            pl.BlockSpec(w1.shape, lambda i: (0, 0)),
            pl.BlockSpec((1, w1.shape[0]), lambda i: (0, 0)),
            pl.BlockSpec(w2.shape, lambda i: (0, 0)),
            pl.BlockSpec((1, w2.shape[0]), lambda i: (0, 0)),
            pl.BlockSpec(w3.shape, lambda i: (0, 0)),
            pl.BlockSpec((1, d3), lambda i: (0, 0)),
        ],
        out_specs=pl.BlockSpec((nb, d3), lambda i: (i, 0)),
        out_shape=jax.ShapeDtypeStruct((n, d3), jnp.float32),
    )(x, w1, b1.reshape(1, -1), w2, b2.reshape(1, -1), w3, b3.reshape(1, -1))


def kernel(x, pos, batch, params):
    x0, p0 = x, pos
    x1, p1 = _block_fwd(params, "b1", x0, p0, 4, _K)
    x2, p2 = _block_fwd(params, "b2", x1, p1, 4, _K)
    x3, p3 = _block_fwd(params, "b3", x2, p2, 4, _K)
    x4, p4 = _block_fwd(params, "b4", x3, p3, 4, _K)
    h = _linear(x4, params["mlp1_W"], params["mlp1_b"], "none")
    h = _fp_fwd(params, "fp4", h, p4, p3, x3)
    h = _fp_fwd(params, "fp3", h, p3, p2, x2)
    h = _fp_fwd(params, "fp2", h, p2, p1, x1)
    h = _fp_fwd(params, "fp1", h, p1, p0, x0)
    lw = jnp.concatenate(
        [params["lin_W"], jnp.zeros((3, params["lin_W"].shape[1]))], axis=0)
    lb = jnp.concatenate([params["lin_b"], jnp.zeros((3,))], axis=0)
    out = _head(h, params["h1_W"], params["h1_b"],
                params["h2_W"], params["h2_b"], lw, lb)
    return out[:, :13]


# speedup vs baseline: 1.8580x; 1.8580x over previous
import functools

import jax
import jax.numpy as jnp
from jax import lax
from jax.experimental import pallas as pl
from jax.experimental.pallas import tpu as pltpu
from jax.experimental.pallas import tpu_sc as plsc

_K = 16


def _lrelu(v, s):
    return jnp.where(v >= 0, v, s * v)


def _knn_body(q_ref, s_ref, o_ref, *, n, st, k):
    qb = q_ref.shape[0]
    qx = q_ref[:, 0:1]
    qy = q_ref[:, 1:2]
    qz = q_ref[:, 2:3]
    it = lax.broadcasted_iota(jnp.int32, (qb, st), 1)
    slot = lax.broadcasted_iota(jnp.int32, (1, k), 1)

    def tile_step(t, carry):
        rv, ri = carry
        sx = s_ref[0:1, pl.ds(t * st, st)]
        sy = s_ref[1:2, pl.ds(t * st, st)]
        sz = s_ref[2:3, pl.ds(t * st, st)]
        d = (qx - sx) ** 2 + (qy - sy) ** 2 + (qz - sz) ** 2

        def round_step(_, carry2):
            d, rv, ri = carry2
            m = jnp.min(d, axis=1, keepdims=True)
            a = jnp.where(d == m, it, n)
            i = jnp.min(a, axis=1, keepdims=True)
            d = jnp.where(it == i, jnp.inf, d)
            w = jnp.max(rv, axis=1, keepdims=True)
            ws = jnp.where(rv == w, slot, k)
            s0 = jnp.min(ws, axis=1, keepdims=True)
            enter = m < w
            sel = (slot == s0) & enter
            rv = jnp.where(sel, m, rv)
            ri = jnp.where(sel, i + t * st, ri)
            return d, rv, ri

        _, rv, ri = lax.fori_loop(0, k, round_step, (d, rv, ri))
        return rv, ri

    rv0 = jnp.full((qb, k), jnp.inf, jnp.float32)
    ri0 = jnp.zeros((qb, k), jnp.int32)
    _, ri = lax.fori_loop(0, n // st, tile_step, (rv0, ri0))
    o_ref[...] = ri


def _knn(q, s_t, k):
    nq = q.shape[0]
    n = s_t.shape[1]
    qb = min(128, nq)
    st = min(2048, n)
    return pl.pallas_call(
        functools.partial(_knn_body, n=n, st=st, k=k),
        grid=(nq // qb,),
        in_specs=[
            pl.BlockSpec((qb, 3), lambda i: (i, 0)),
            pl.BlockSpec((3, n), lambda i: (0, 0)),
        ],
        out_specs=pl.BlockSpec((qb, k), lambda i: (i, 0)),
        out_shape=jax.ShapeDtypeStruct((nq, k), jnp.int32),
    )(q, s_t)


def _nn1_body(q_ref, s_ref, o_ref, *, n, st):
    qb = q_ref.shape[0]
    qx = q_ref[:, 0:1]
    qy = q_ref[:, 1:2]
    qz = q_ref[:, 2:3]
    it = lax.broadcasted_iota(jnp.int32, (qb, st), 1)

    def tile_step(t, carry):
        bv, bi = carry
        sx = s_ref[0:1, pl.ds(t * st, st)]
        sy = s_ref[1:2, pl.ds(t * st, st)]
        sz = s_ref[2:3, pl.ds(t * st, st)]
        d = (qx - sx) ** 2 + (qy - sy) ** 2 + (qz - sz) ** 2
        m = jnp.min(d, axis=1, keepdims=True)
        a = jnp.where(d == m, it, n)
        i = jnp.min(a, axis=1, keepdims=True) + t * st
        better = m < bv
        bv = jnp.where(better, m, bv)
        bi = jnp.where(better, i, bi)
        return bv, bi

    bv0 = jnp.full((qb, 1), jnp.inf, jnp.float32)
    bi0 = jnp.zeros((qb, 1), jnp.int32)
    _, bi = lax.fori_loop(0, n // st, tile_step, (bv0, bi0))
    o_ref[...] = bi


def _nn1(q, s_t):
    nq = q.shape[0]
    n = s_t.shape[1]
    qb = min(128, nq)
    st = min(2048, n)
    out = pl.pallas_call(
        functools.partial(_nn1_body, n=n, st=st),
        grid=(nq // qb,),
        in_specs=[
            pl.BlockSpec((qb, 3), lambda i: (i, 0)),
            pl.BlockSpec((3, n), lambda i: (0, 0)),
        ],
        out_specs=pl.BlockSpec((qb, 1), lambda i: (i, 0)),
        out_shape=jax.ShapeDtypeStruct((nq, 1), jnp.int32),
    )(q, s_t)
    return out[:, 0]


def _lin_body(x_ref, w_ref, b_ref, o_ref, *, act, slope):
    y = lax.dot_general(x_ref[...], w_ref[...], (((1,), (1,)), ((), ())),
                        preferred_element_type=jnp.float32)
    y = y + b_ref[...]
    if act == "lrelu":
        y = _lrelu(y, slope)
    elif act == "relu":
        y = jnp.maximum(y, 0.0)
    o_ref[...] = y


def _lin_res_body(x_ref, w_ref, b_ref, r_ref, o_ref, *, slope, slope2):
    y = lax.dot_general(x_ref[...], w_ref[...], (((1,), (1,)), ((), ())),
                        preferred_element_type=jnp.float32)
    y = _lrelu(y + b_ref[...], slope)
    o_ref[...] = _lrelu(y + r_ref[...], slope2)


def _linear(x, w, b, act="none", slope=0.2):
    n, din = x.shape
    dout = w.shape[0]
    nb = min(512, n)
    return pl.pallas_call(
        functools.partial(_lin_body, act=act, slope=slope),
        grid=(n // nb,),
        in_specs=[
            pl.BlockSpec((nb, din), lambda i: (i, 0)),
            pl.BlockSpec((dout, din), lambda i: (0, 0)),
            pl.BlockSpec((1, dout), lambda i: (0, 0)),
        ],
        out_specs=pl.BlockSpec((nb, dout), lambda i: (i, 0)),
        out_shape=jax.ShapeDtypeStruct((n, dout), jnp.float32),
    )(x, w, b.reshape(1, -1))


def _linear_res(x, w, b, res, slope=0.2, slope2=0.01):
    n, din = x.shape
    dout = w.shape[0]
    nb = min(512, n)
    return pl.pallas_call(
        functools.partial(_lin_res_body, slope=slope, slope2=slope2),
        grid=(n // nb,),
        in_specs=[
            pl.BlockSpec((nb, din), lambda i: (i, 0)),
            pl.BlockSpec((dout, din), lambda i: (0, 0)),
            pl.BlockSpec((1, dout), lambda i: (0, 0)),
            pl.BlockSpec((nb, dout), lambda i: (i, 0)),
        ],
        out_specs=pl.BlockSpec((nb, dout), lambda i: (i, 0)),
        out_shape=jax.ShapeDtypeStruct((n, dout), jnp.float32),
    )(x, w, b.reshape(1, -1), res)


def _lfa_body(xj_ref, pi_ref, pj_ref, ew_ref, eb_ref, aw_ref, ab_ref, o_ref,
              *, k, qb):
    pi = pi_ref[...]
    pj = pj_ref[...]
    dist = pj - pi
    eu = jnp.sum(jnp.abs(dist), axis=1, keepdims=True)
    rel = jnp.concatenate([pi, pj, dist, eu], axis=1)
    lse = lax.dot_general(rel, ew_ref[...], (((1,), (1,)), ((), ())),
                          preferred_element_type=jnp.float32) + eb_ref[...]
    out1 = jnp.concatenate([xj_ref[...], lse], axis=1)
    logits = lax.dot_general(out1, aw_ref[...], (((1,), (1,)), ((), ())),
                             preferred_element_type=jnp.float32) + ab_ref[...]
    z = logits - jnp.max(logits, axis=1, keepdims=True)
    e = jnp.exp(z)
    att = e / jnp.sum(e, axis=1, keepdims=True)
    msg = att * out1
    eb = qb * k
    rows = lax.broadcasted_iota(jnp.int32, (qb, eb), 0)
    cols = lax.broadcasted_iota(jnp.int32, (qb, eb), 1)
    sel = jnp.where(cols // k == rows, 1.0, 0.0)
    o_ref[...] = lax.dot_general(sel, msg, (((1,), (0,)), ((), ())),
                                 preferred_element_type=jnp.float32)


def _lfa(xj, pos_i, pos_j, ew, ebias, aw, abias, k):
    e, c1 = xj.shape
    nq = e // k
    d = aw.shape[0]
    qb = min(64 if d >= 256 else 128, nq)
    eb = qb * k
    return pl.pallas_call(
        functools.partial(_lfa_body, k=k, qb=qb),
        grid=(nq // qb,),
        in_specs=[
            pl.BlockSpec((eb, c1), lambda i: (i, 0)),
            pl.BlockSpec((eb, 3), lambda i: (i, 0)),
            pl.BlockSpec((eb, 3), lambda i: (i, 0)),
            pl.BlockSpec(ew.shape, lambda i: (0, 0)),
            pl.BlockSpec((1, ew.shape[0]), lambda i: (0, 0)),
            pl.BlockSpec(aw.shape, lambda i: (0, 0)),
            pl.BlockSpec((1, d), lambda i: (0, 0)),
        ],
        out_specs=pl.BlockSpec((qb, d), lambda i: (i, 0)),
        out_shape=jax.ShapeDtypeStruct((nq, d), jnp.float32),
    )(xj, pos_i, pos_j, ew, ebias.reshape(1, -1), aw, abias.reshape(1, -1))


def _sc_gather(table, idx):
    v, dd = table.shape
    b = idx.shape[0]
    info = plsc.get_sparse_core_info()
    nw = info.num_cores * info.num_subcores
    bpw = b // nw
    ch = min(128, bpw)
    nch = bpw // ch
    mesh = plsc.VectorSubcoreMesh(core_axis_name="c", subcore_axis_name="s")

    @functools.partial(
        pl.kernel, mesh=mesh,
        compiler_params=pltpu.CompilerParams(use_tc_tiling_on_sc=False),
        out_type=jax.ShapeDtypeStruct((b, dd), jnp.float32),
        scratch_types=[
            pltpu.VMEM((bpw,), jnp.int32),
            pltpu.VMEM((bpw, dd), jnp.float32),
            pltpu.SemaphoreType.DMA,
        ],
    )
    def gk(table_hbm, idx_hbm, out_hbm, idx_v, rows_v, sem):
        wid = lax.axis_index("s") * info.num_cores + lax.axis_index("c")
        base = wid * bpw
        pltpu.sync_copy(idx_hbm.at[pl.ds(base, bpw)], idx_v)

        def body(j, carry):
            off = pl.multiple_of(j * ch, 8)
            pltpu.async_copy(table_hbm.at[idx_v.at[pl.ds(off, ch)]],
                             rows_v.at[pl.ds(off, ch)], sem).wait()
            return carry

        lax.fori_loop(0, nch, body, 0)
        pltpu.sync_copy(rows_v, out_hbm.at[pl.ds(base, bpw)])

    return gk(table, idx)


def _pad_cols(a, mult):
    n, c = a.shape
    pad = (-c) % mult
    if pad:
        a = jnp.concatenate([a, jnp.zeros((n, pad), a.dtype)], axis=1)
    return a


def _gather_rows(table, idx, want_cols):
    tp = _pad_cols(table, 16)
    return _sc_gather(tp, idx)[:, :want_cols]


def _block_fwd(params, pfx, x, pos, dec, k):
    n = x.shape[0]
    nq = n // dec
    q = pos[::dec]
    nbrs = _knn(q, pos.T, k)
    col = nbrs.reshape(-1)
    sc = _linear(x, params[pfx + "_sc_W"], params[pfx + "_sc_b"], "lrelu", 0.2)
    hm = _linear(x, params[pfx + "_m1_W"], params[pfx + "_m1_b"], "lrelu", 0.2)
    pos_j = _gather_rows(pos, col, 3)
    pos_i = jnp.repeat(pos[:nq], k, axis=0)
    xj = _gather_rows(hm, col, hm.shape[1])
    h1q = _lfa(xj, pos_i, pos_j,
               params[pfx + "_l1_e_W"], params[pfx + "_l1_e_b"],
               params[pfx + "_l1_a_W"], params[pfx + "_l1_a_b"], k)
    d1 = h1q.shape[1]
    h1full = jnp.concatenate(
        [h1q, jnp.zeros((n - nq, d1), jnp.float32)], axis=0)
    xj2 = _gather_rows(h1full, col, d1)
    h2q = _lfa(xj2, pos_i, pos_j,
               params[pfx + "_l2_e_W"], params[pfx + "_l2_e_b"],
               params[pfx + "_l2_a_W"], params[pfx + "_l2_a_b"], k)
    d2 = h2q.shape[1]
    h2full = jnp.concatenate(
        [h2q, jnp.zeros((n - nq, d2), jnp.float32)], axis=0)
    h4 = _linear_res(h2full, params[pfx + "_m2_W"], params[pfx + "_m2_b"], sc)
    return h4[::dec], q


def _fp_fwd(params, pfx, xh, pos, pos_skip, x_skip):
    nn = _nn1(pos_skip, pos.T)
    xi = _gather_rows(xh, nn, xh.shape[1])
    if x_skip is not None:
        xi = jnp.concatenate([xi, x_skip], axis=1)
    return _linear(xi, params[pfx + "_W"], params[pfx + "_b"], "none")


def _head_body(x_ref, w1_ref, b1_ref, w2_ref, b2_ref, w3_ref, b3_ref, o_ref):
    h = lax.dot_general(x_ref[...], w1_ref[...], (((1,), (1,)), ((), ())),
                        preferred_element_type=jnp.float32) + b1_ref[...]
    h = jnp.maximum(h, 0.0)
    h = lax.dot_general(h, w2_ref[...], (((1,), (1,)), ((), ())),
                        preferred_element_type=jnp.float32) + b2_ref[...]
    h = lax.dot_general(h, w3_ref[...], (((1,), (1,)), ((), ())),
                        preferred_element_type=jnp.float32) + b3_ref[...]
    o_ref[...] = h


def _head(x, w1, b1, w2, b2, w3, b3):
    n = x.shape[0]
    nb = min(512, n)
    d3 = w3.shape[0]
    return pl.pallas_call(
        _head_body,
        grid=(n // nb,),
        in_specs=[
            pl.BlockSpec((nb, x.shape[1]), lambda i: (i, 0)),
            pl.BlockSpec(w1.shape, lambda i: (0, 0)),
            pl.BlockSpec((1, w1.shape[0]), lambda i: (0, 0)),
            pl.BlockSpec(w2.shape, lambda i: (0, 0)),
            pl.BlockSpec((1, w2.shape[0]), lambda i: (0, 0)),
            pl.BlockSpec(w3.shape, lambda i: (0, 0)),
            pl.BlockSpec((1, d3), lambda i: (0, 0)),
        ],
        out_specs=pl.BlockSpec((nb, d3), lambda i: (i, 0)),
        out_shape=jax.ShapeDtypeStruct((n, d3), jnp.float32),
    )(x, w1, b1.reshape(1, -1), w2, b2.reshape(1, -1), w3, b3.reshape(1, -1))


def kernel(x, pos, batch, params):
    x0, p0 = x, pos
    x1, p1 = _block_fwd(params, "b1", x0, p0, 4, _K)
    x2, p2 = _block_fwd(params, "b2", x1, p1, 4, _K)
    x3, p3 = _block_fwd(params, "b3", x2, p2, 4, _K)
    x4, p4 = _block_fwd(params, "b4", x3, p3, 4, _K)
    h = _linear(x4, params["mlp1_W"], params["mlp1_b"], "none")
    h = _fp_fwd(params, "fp4", h, p4, p3, x3)
    h = _fp_fwd(params, "fp3", h, p3, p2, x2)
    h = _fp_fwd(params, "fp2", h, p2, p1, x1)
    h = _fp_fwd(params, "fp1", h, p1, p0, x0)
    lw = jnp.concatenate(
        [params["lin_W"], jnp.zeros((3, params["lin_W"].shape[1]))], axis=0)
    lb = jnp.concatenate([params["lin_b"], jnp.zeros((3,))], axis=0)
    out = _head(h, params["h1_W"], params["h1_b"],
                params["h2_W"], params["h2_b"], lw, lb)
    return out[:, :13]


# speedup vs baseline: 2.6168x; 1.4084x over previous
import functools

import jax
import jax.numpy as jnp
from jax import lax
from jax.experimental import pallas as pl
from jax.experimental.pallas import tpu as pltpu
from jax.experimental.pallas import tpu_sc as plsc

_K = 16


def _lrelu(v, s):
    return jnp.where(v >= 0, v, s * v)


def _knn_body(q_ref, s_ref, o_ref, *, n, st, k):
    qb = q_ref.shape[0]
    qx = q_ref[:, 0:1]
    qy = q_ref[:, 1:2]
    qz = q_ref[:, 2:3]
    it = lax.broadcasted_iota(jnp.int32, (qb, st), 1)
    slot = lax.broadcasted_iota(jnp.int32, (1, k), 1)

    def tile_step(t, carry):
        rv, ri = carry
        sx = s_ref[0:1, pl.ds(t * st, st)]
        sy = s_ref[1:2, pl.ds(t * st, st)]
        sz = s_ref[2:3, pl.ds(t * st, st)]
        d = (qx - sx) ** 2 + (qy - sy) ** 2 + (qz - sz) ** 2

        w0 = jnp.max(rv, axis=1, keepdims=True)
        m0 = jnp.min(d, axis=1, keepdims=True)
        go0 = jnp.any(m0 < w0)

        def cond(c):
            return c[0]

        def round_step(c):
            _, d, m, rv, ri, w = c
            a = jnp.where(d == m, it, n)
            i = jnp.min(a, axis=1, keepdims=True)
            d = jnp.where(it == i, jnp.inf, d)
            ws = jnp.where(rv == w, slot, k)
            s0 = jnp.min(ws, axis=1, keepdims=True)
            sel = (slot == s0) & (m < w)
            rv = jnp.where(sel, m, rv)
            ri = jnp.where(sel, i + t * st, ri)
            w = jnp.max(rv, axis=1, keepdims=True)
            m = jnp.min(d, axis=1, keepdims=True)
            return jnp.any(m < w), d, m, rv, ri, w

        _, _, _, rv, ri, _ = lax.while_loop(
            cond, round_step, (go0, d, m0, rv, ri, w0))
        return rv, ri

    rv0 = jnp.full((qb, k), jnp.inf, jnp.float32)
    ri0 = jnp.zeros((qb, k), jnp.int32)
    _, ri = lax.fori_loop(0, n // st, tile_step, (rv0, ri0))
    o_ref[...] = ri


def _knn(q, s_t, k):
    nq = q.shape[0]
    n = s_t.shape[1]
    qb = min(128, nq)
    st = min(1024, n)
    return pl.pallas_call(
        functools.partial(_knn_body, n=n, st=st, k=k),
        grid=(nq // qb,),
        in_specs=[
            pl.BlockSpec((qb, 3), lambda i: (i, 0)),
            pl.BlockSpec((3, n), lambda i: (0, 0)),
        ],
        out_specs=pl.BlockSpec((qb, k), lambda i: (i, 0)),
        out_shape=jax.ShapeDtypeStruct((nq, k), jnp.int32),
    )(q, s_t)


def _nn1_body(q_ref, s_ref, o_ref, *, n, st):
    qb = q_ref.shape[0]
    qx = q_ref[:, 0:1]
    qy = q_ref[:, 1:2]
    qz = q_ref[:, 2:3]
    it = lax.broadcasted_iota(jnp.int32, (qb, st), 1)

    def tile_step(t, carry):
        bv, bi = carry
        sx = s_ref[0:1, pl.ds(t * st, st)]
        sy = s_ref[1:2, pl.ds(t * st, st)]
        sz = s_ref[2:3, pl.ds(t * st, st)]
        d = (qx - sx) ** 2 + (qy - sy) ** 2 + (qz - sz) ** 2
        m = jnp.min(d, axis=1, keepdims=True)
        a = jnp.where(d == m, it, n)
        i = jnp.min(a, axis=1, keepdims=True) + t * st
        better = m < bv
        bv = jnp.where(better, m, bv)
        bi = jnp.where(better, i, bi)
        return bv, bi

    bv0 = jnp.full((qb, 1), jnp.inf, jnp.float32)
    bi0 = jnp.zeros((qb, 1), jnp.int32)
    _, bi = lax.fori_loop(0, n // st, tile_step, (bv0, bi0))
    o_ref[...] = bi


def _nn1(q, s_t):
    nq = q.shape[0]
    n = s_t.shape[1]
    qb = min(128, nq)
    st = min(2048, n)
    out = pl.pallas_call(
        functools.partial(_nn1_body, n=n, st=st),
        grid=(nq // qb,),
        in_specs=[
            pl.BlockSpec((qb, 3), lambda i: (i, 0)),
            pl.BlockSpec((3, n), lambda i: (0, 0)),
        ],
        out_specs=pl.BlockSpec((qb, 1), lambda i: (i, 0)),
        out_shape=jax.ShapeDtypeStruct((nq, 1), jnp.int32),
    )(q, s_t)
    return out[:, 0]


def _lin_body(x_ref, w_ref, b_ref, o_ref, *, act, slope):
    y = lax.dot_general(x_ref[...], w_ref[...], (((1,), (1,)), ((), ())),
                        preferred_element_type=jnp.float32)
    y = y + b_ref[...]
    if act == "lrelu":
        y = _lrelu(y, slope)
    elif act == "relu":
        y = jnp.maximum(y, 0.0)
    o_ref[...] = y


def _lin_res_body(x_ref, w_ref, b_ref, r_ref, o_ref, *, slope, slope2):
    y = lax.dot_general(x_ref[...], w_ref[...], (((1,), (1,)), ((), ())),
                        preferred_element_type=jnp.float32)
    y = _lrelu(y + b_ref[...], slope)
    o_ref[...] = _lrelu(y + r_ref[...], slope2)


def _linear(x, w, b, act="none", slope=0.2):
    n, din = x.shape
    dout = w.shape[0]
    nb = min(512, n)
    return pl.pallas_call(
        functools.partial(_lin_body, act=act, slope=slope),
        grid=(n // nb,),
        in_specs=[
            pl.BlockSpec((nb, din), lambda i: (i, 0)),
            pl.BlockSpec((dout, din), lambda i: (0, 0)),
            pl.BlockSpec((1, dout), lambda i: (0, 0)),
        ],
        out_specs=pl.BlockSpec((nb, dout), lambda i: (i, 0)),
        out_shape=jax.ShapeDtypeStruct((n, dout), jnp.float32),
    )(x, w, b.reshape(1, -1))


def _linear_res(x, w, b, res, slope=0.2, slope2=0.01):
    n, din = x.shape
    dout = w.shape[0]
    nb = min(512, n)
    return pl.pallas_call(
        functools.partial(_lin_res_body, slope=slope, slope2=slope2),
        grid=(n // nb,),
        in_specs=[
            pl.BlockSpec((nb, din), lambda i: (i, 0)),
            pl.BlockSpec((dout, din), lambda i: (0, 0)),
            pl.BlockSpec((1, dout), lambda i: (0, 0)),
            pl.BlockSpec((nb, dout), lambda i: (i, 0)),
        ],
        out_specs=pl.BlockSpec((nb, dout), lambda i: (i, 0)),
        out_shape=jax.ShapeDtypeStruct((n, dout), jnp.float32),
    )(x, w, b.reshape(1, -1), res)


def _lfa_body(xj_ref, pi_ref, pj_ref, ew_ref, eb_ref, aw_ref, ab_ref, o_ref,
              *, k, qb):
    pi = pi_ref[...]
    pj = pj_ref[...]
    dist = pj - pi
    eu = jnp.sum(jnp.abs(dist), axis=1, keepdims=True)
    rel = jnp.concatenate([pi, pj, dist, eu], axis=1)
    lse = lax.dot_general(rel, ew_ref[...], (((1,), (1,)), ((), ())),
                          preferred_element_type=jnp.float32) + eb_ref[...]
    out1 = jnp.concatenate([xj_ref[...], lse], axis=1)
    logits = lax.dot_general(out1, aw_ref[...], (((1,), (1,)), ((), ())),
                             preferred_element_type=jnp.float32) + ab_ref[...]
    z = logits - jnp.max(logits, axis=1, keepdims=True)
    e = jnp.exp(z)
    att = e / jnp.sum(e, axis=1, keepdims=True)
    msg = att * out1
    eb = qb * k
    rows = lax.broadcasted_iota(jnp.int32, (qb, eb), 0)
    cols = lax.broadcasted_iota(jnp.int32, (qb, eb), 1)
    sel = jnp.where(cols // k == rows, 1.0, 0.0)
    o_ref[...] = lax.dot_general(sel, msg, (((1,), (0,)), ((), ())),
                                 preferred_element_type=jnp.float32)


def _lfa(xj, pos_i, pos_j, ew, ebias, aw, abias, k):
    e, c1 = xj.shape
    nq = e // k
    d = aw.shape[0]
    qb = min(64 if d >= 256 else 128, nq)
    eb = qb * k
    return pl.pallas_call(
        functools.partial(_lfa_body, k=k, qb=qb),
        grid=(nq // qb,),
        in_specs=[
            pl.BlockSpec((eb, c1), lambda i: (i, 0)),
            pl.BlockSpec((eb, 3), lambda i: (i, 0)),
            pl.BlockSpec((eb, 3), lambda i: (i, 0)),
            pl.BlockSpec(ew.shape, lambda i: (0, 0)),
            pl.BlockSpec((1, ew.shape[0]), lambda i: (0, 0)),
            pl.BlockSpec(aw.shape, lambda i: (0, 0)),
            pl.BlockSpec((1, d), lambda i: (0, 0)),
        ],
        out_specs=pl.BlockSpec((qb, d), lambda i: (i, 0)),
        out_shape=jax.ShapeDtypeStruct((nq, d), jnp.float32),
    )(xj, pos_i, pos_j, ew, ebias.reshape(1, -1), aw, abias.reshape(1, -1))


def _sc_gather(table, idx):
    v, dd = table.shape
    b = idx.shape[0]
    info = plsc.get_sparse_core_info()
    nw = info.num_cores * info.num_subcores
    bpw = b // nw
    ch = min(128, bpw)
    nch = bpw // ch
    mesh = plsc.VectorSubcoreMesh(core_axis_name="c", subcore_axis_name="s")

    @functools.partial(
        pl.kernel, mesh=mesh,
        compiler_params=pltpu.CompilerParams(use_tc_tiling_on_sc=False),
        out_type=jax.ShapeDtypeStruct((b, dd), jnp.float32),
        scratch_types=[
            pltpu.VMEM((bpw,), jnp.int32),
            pltpu.VMEM((bpw, dd), jnp.float32),
            pltpu.SemaphoreType.DMA,
        ],
    )
    def gk(table_hbm, idx_hbm, out_hbm, idx_v, rows_v, sem):
        wid = lax.axis_index("s") * info.num_cores + lax.axis_index("c")
        base = wid * bpw
        pltpu.sync_copy(idx_hbm.at[pl.ds(base, bpw)], idx_v)

        def body(j, carry):
            off = pl.multiple_of(j * ch, 8)
            pltpu.async_copy(table_hbm.at[idx_v.at[pl.ds(off, ch)]],
                             rows_v.at[pl.ds(off, ch)], sem).wait()
            return carry

        lax.fori_loop(0, nch, body, 0)
        pltpu.sync_copy(rows_v, out_hbm.at[pl.ds(base, bpw)])

    return gk(table, idx)


def _pad_cols(a, mult):
    n, c = a.shape
    pad = (-c) % mult
    if pad:
        a = jnp.concatenate([a, jnp.zeros((n, pad), a.dtype)], axis=1)
    return a


def _gather_rows(table, idx, want_cols):
    tp = _pad_cols(table, 16)
    return _sc_gather(tp, idx)[:, :want_cols]


def _block_fwd(params, pfx, x, pos, dec, k):
    n = x.shape[0]
    nq = n // dec
    q = pos[::dec]
    nbrs = _knn(q, pos.T, k)
    col = nbrs.reshape(-1)
    sc = _linear(x, params[pfx + "_sc_W"], params[pfx + "_sc_b"], "lrelu", 0.2)
    hm = _linear(x, params[pfx + "_m1_W"], params[pfx + "_m1_b"], "lrelu", 0.2)
    pos_j = _gather_rows(pos, col, 3)
    pos_i = jnp.repeat(pos[:nq], k, axis=0)
    xj = _gather_rows(hm, col, hm.shape[1])
    h1q = _lfa(xj, pos_i, pos_j,
               params[pfx + "_l1_e_W"], params[pfx + "_l1_e_b"],
               params[pfx + "_l1_a_W"], params[pfx + "_l1_a_b"], k)
    d1 = h1q.shape[1]
    h1full = jnp.concatenate(
        [h1q, jnp.zeros((n - nq, d1), jnp.float32)], axis=0)
    xj2 = _gather_rows(h1full, col, d1)
    h2q = _lfa(xj2, pos_i, pos_j,
               params[pfx + "_l2_e_W"], params[pfx + "_l2_e_b"],
               params[pfx + "_l2_a_W"], params[pfx + "_l2_a_b"], k)
    d2 = h2q.shape[1]
    h2full = jnp.concatenate(
        [h2q, jnp.zeros((n - nq, d2), jnp.float32)], axis=0)
    h4 = _linear_res(h2full, params[pfx + "_m2_W"], params[pfx + "_m2_b"], sc)
    return h4[::dec], q


def _fp_fwd(params, pfx, xh, pos, pos_skip, x_skip):
    nn = _nn1(pos_skip, pos.T)
    xi = _gather_rows(xh, nn, xh.shape[1])
    if x_skip is not None:
        xi = jnp.concatenate([xi, x_skip], axis=1)
    return _linear(xi, params[pfx + "_W"], params[pfx + "_b"], "none")


def _head_body(x_ref, w1_ref, b1_ref, w2_ref, b2_ref, w3_ref, b3_ref, o_ref):
    h = lax.dot_general(x_ref[...], w1_ref[...], (((1,), (1,)), ((), ())),
                        preferred_element_type=jnp.float32) + b1_ref[...]
    h = jnp.maximum(h, 0.0)
    h = lax.dot_general(h, w2_ref[...], (((1,), (1,)), ((), ())),
                        preferred_element_type=jnp.float32) + b2_ref[...]
    h = lax.dot_general(h, w3_ref[...], (((1,), (1,)), ((), ())),
                        preferred_element_type=jnp.float32) + b3_ref[...]
    o_ref[...] = h


def _head(x, w1, b1, w2, b2, w3, b3):
    n = x.shape[0]
    nb = min(512, n)
    d3 = w3.shape[0]
    return pl.pallas_call(
        _head_body,
        grid=(n // nb,),
        in_specs=[
            pl.BlockSpec((nb, x.shape[1]), lambda i: (i, 0)),
            pl.BlockSpec(w1.shape, lambda i: (0, 0)),
            pl.BlockSpec((1, w1.shape[0]), lambda i: (0, 0)),
            pl.BlockSpec(w2.shape, lambda i: (0, 0)),
            pl.BlockSpec((1, w2.shape[0]), lambda i: (0, 0)),
            pl.BlockSpec(w3.shape, lambda i: (0, 0)),
            pl.BlockSpec((1, d3), lambda i: (0, 0)),
        ],
        out_specs=pl.BlockSpec((nb, d3), lambda i: (i, 0)),
        out_shape=jax.ShapeDtypeStruct((n, d3), jnp.float32),
    )(x, w1, b1.reshape(1, -1), w2, b2.reshape(1, -1), w3, b3.reshape(1, -1))


def kernel(x, pos, batch, params):
    x0, p0 = x, pos
    x1, p1 = _block_fwd(params, "b1", x0, p0, 4, _K)
    x2, p2 = _block_fwd(params, "b2", x1, p1, 4, _K)
    x3, p3 = _block_fwd(params, "b3", x2, p2, 4, _K)
    x4, p4 = _block_fwd(params, "b4", x3, p3, 4, _K)
    h = _linear(x4, params["mlp1_W"], params["mlp1_b"], "none")
    h = _fp_fwd(params, "fp4", h, p4, p3, x3)
    h = _fp_fwd(params, "fp3", h, p3, p2, x2)
    h = _fp_fwd(params, "fp2", h, p2, p1, x1)
    h = _fp_fwd(params, "fp1", h, p1, p0, x0)
    lw = jnp.concatenate(
        [params["lin_W"], jnp.zeros((3, params["lin_W"].shape[1]))], axis=0)
    lb = jnp.concatenate([params["lin_b"], jnp.zeros((3,))], axis=0)
    out = _head(h, params["h1_W"], params["h1_b"],
                params["h2_W"], params["h2_b"], lw, lb)
    return out[:, :13]


# speedup vs baseline: 2.7799x; 1.0623x over previous
import functools

import jax
import jax.numpy as jnp
from jax import lax
from jax.experimental import pallas as pl
from jax.experimental.pallas import tpu as pltpu
from jax.experimental.pallas import tpu_sc as plsc

_K = 16


def _lrelu(v, s):
    return jnp.where(v >= 0, v, s * v)


def _knn_body(q_ref, s_ref, o_ref, *, n, st, k):
    qb = q_ref.shape[1]
    qx = q_ref[0:1, :]
    qy = q_ref[1:2, :]
    qz = q_ref[2:3, :]
    it = lax.broadcasted_iota(jnp.int32, (st, qb), 0)
    slot = lax.broadcasted_iota(jnp.int32, (k, 1), 0)

    def tile_step(t, carry):
        rv, ri = carry
        sx = s_ref[pl.ds(t * st, st), 0:1]
        sy = s_ref[pl.ds(t * st, st), 1:2]
        sz = s_ref[pl.ds(t * st, st), 2:3]
        d = (sx - qx) ** 2 + (sy - qy) ** 2 + (sz - qz) ** 2

        w0 = jnp.max(rv, axis=0, keepdims=True)
        m0 = jnp.min(d, axis=0, keepdims=True)
        go0 = jnp.any(m0 < w0)

        def cond(c):
            return c[0]

        def round_step(c):
            _, d, m, rv, ri, w = c
            a = jnp.where(d == m, it, n)
            i = jnp.min(a, axis=0, keepdims=True)
            d = jnp.where(it == i, jnp.inf, d)
            ws = jnp.where(rv == w, slot, k)
            s0 = jnp.min(ws, axis=0, keepdims=True)
            sel = (slot == s0) & (m < w)
            rv = jnp.where(sel, m, rv)
            ri = jnp.where(sel, i + t * st, ri)
            w = jnp.max(rv, axis=0, keepdims=True)
            m = jnp.min(d, axis=0, keepdims=True)
            return jnp.any(m < w), d, m, rv, ri, w

        _, _, _, rv, ri, _ = lax.while_loop(
            cond, round_step, (go0, d, m0, rv, ri, w0))
        return rv, ri

    rv0 = jnp.full((k, qb), jnp.inf, jnp.float32)
    ri0 = jnp.zeros((k, qb), jnp.int32)
    _, ri = lax.fori_loop(0, n // st, tile_step, (rv0, ri0))
    o_ref[...] = ri


def _knn(q_t, s, k):
    nq = q_t.shape[1]
    n = s.shape[0]
    qb = min(128, nq)
    st = min(1024, n)
    return pl.pallas_call(
        functools.partial(_knn_body, n=n, st=st, k=k),
        grid=(nq // qb,),
        in_specs=[
            pl.BlockSpec((3, qb), lambda i: (0, i)),
            pl.BlockSpec((n, 3), lambda i: (0, 0)),
        ],
        out_specs=pl.BlockSpec((k, qb), lambda i: (0, i)),
        out_shape=jax.ShapeDtypeStruct((k, nq), jnp.int32),
    )(q_t, s)


def _nn1_body(q_ref, s_ref, o_ref, *, n, st):
    qb = q_ref.shape[1]
    qx = q_ref[0:1, :]
    qy = q_ref[1:2, :]
    qz = q_ref[2:3, :]
    it = lax.broadcasted_iota(jnp.int32, (st, qb), 0)

    def tile_step(t, carry):
        bv, bi = carry
        sx = s_ref[pl.ds(t * st, st), 0:1]
        sy = s_ref[pl.ds(t * st, st), 1:2]
        sz = s_ref[pl.ds(t * st, st), 2:3]
        d = (sx - qx) ** 2 + (sy - qy) ** 2 + (sz - qz) ** 2
        m = jnp.min(d, axis=0, keepdims=True)
        a = jnp.where(d == m, it, n)
        i = jnp.min(a, axis=0, keepdims=True) + t * st
        better = m < bv
        bv = jnp.where(better, m, bv)
        bi = jnp.where(better, i, bi)
        return bv, bi

    bv0 = jnp.full((1, qb), jnp.inf, jnp.float32)
    bi0 = jnp.zeros((1, qb), jnp.int32)
    _, bi = lax.fori_loop(0, n // st, tile_step, (bv0, bi0))
    o_ref[...] = bi


def _nn1(q_t, s):
    nq = q_t.shape[1]
    n = s.shape[0]
    qb = min(128, nq)
    st = min(2048, n)
    out = pl.pallas_call(
        functools.partial(_nn1_body, n=n, st=st),
        grid=(nq // qb,),
        in_specs=[
            pl.BlockSpec((3, qb), lambda i: (0, i)),
            pl.BlockSpec((n, 3), lambda i: (0, 0)),
        ],
        out_specs=pl.BlockSpec((1, qb), lambda i: (0, i)),
        out_shape=jax.ShapeDtypeStruct((1, nq), jnp.int32),
    )(q_t, s)
    return out[0]


def _lin_body(x_ref, w_ref, b_ref, o_ref, *, act, slope):
    y = lax.dot_general(x_ref[...], w_ref[...], (((1,), (1,)), ((), ())),
                        preferred_element_type=jnp.float32)
    y = y + b_ref[...]
    if act == "lrelu":
        y = _lrelu(y, slope)
    elif act == "relu":
        y = jnp.maximum(y, 0.0)
    o_ref[...] = y


def _lin_res_body(x_ref, w_ref, b_ref, r_ref, o_ref, *, slope, slope2):
    y = lax.dot_general(x_ref[...], w_ref[...], (((1,), (1,)), ((), ())),
                        preferred_element_type=jnp.float32)
    y = _lrelu(y + b_ref[...], slope)
    o_ref[...] = _lrelu(y + r_ref[...], slope2)


def _linear(x, w, b, act="none", slope=0.2):
    n, din = x.shape
    dout = w.shape[0]
    nb = min(512, n)
    return pl.pallas_call(
        functools.partial(_lin_body, act=act, slope=slope),
        grid=(n // nb,),
        in_specs=[
            pl.BlockSpec((nb, din), lambda i: (i, 0)),
            pl.BlockSpec((dout, din), lambda i: (0, 0)),
            pl.BlockSpec((1, dout), lambda i: (0, 0)),
        ],
        out_specs=pl.BlockSpec((nb, dout), lambda i: (i, 0)),
        out_shape=jax.ShapeDtypeStruct((n, dout), jnp.float32),
    )(x, w, b.reshape(1, -1))


def _linear_res(x, w, b, res, slope=0.2, slope2=0.01):
    n, din = x.shape
    dout = w.shape[0]
    nb = min(512, n)
    return pl.pallas_call(
        functools.partial(_lin_res_body, slope=slope, slope2=slope2),
        grid=(n // nb,),
        in_specs=[
            pl.BlockSpec((nb, din), lambda i: (i, 0)),
            pl.BlockSpec((dout, din), lambda i: (0, 0)),
            pl.BlockSpec((1, dout), lambda i: (0, 0)),
            pl.BlockSpec((nb, dout), lambda i: (i, 0)),
        ],
        out_specs=pl.BlockSpec((nb, dout), lambda i: (i, 0)),
        out_shape=jax.ShapeDtypeStruct((n, dout), jnp.float32),
    )(x, w, b.reshape(1, -1), res)


def _lfa_body(xj_ref, pi_ref, pj_ref, ew_ref, eb_ref, aw_ref, ab_ref, o_ref,
              *, k, qb):
    pi = pi_ref[...]
    pj = pj_ref[...]
    dist = pj - pi
    eu = jnp.sum(jnp.abs(dist), axis=1, keepdims=True)
    rel = jnp.concatenate([pi, pj, dist, eu], axis=1)
    lse = lax.dot_general(rel, ew_ref[...], (((1,), (1,)), ((), ())),
                          preferred_element_type=jnp.float32) + eb_ref[...]
    out1 = jnp.concatenate([xj_ref[...], lse], axis=1)
    logits = lax.dot_general(out1, aw_ref[...], (((1,), (1,)), ((), ())),
                             preferred_element_type=jnp.float32) + ab_ref[...]
    z = logits - jnp.max(logits, axis=1, keepdims=True)
    e = jnp.exp(z)
    att = e / jnp.sum(e, axis=1, keepdims=True)
    msg = att * out1
    eb = qb * k
    rows = lax.broadcasted_iota(jnp.int32, (qb, eb), 0)
    cols = lax.broadcasted_iota(jnp.int32, (qb, eb), 1)
    sel = jnp.where(cols // k == rows, 1.0, 0.0)
    o_ref[...] = lax.dot_general(sel, msg, (((1,), (0,)), ((), ())),
                                 preferred_element_type=jnp.float32)


def _lfa(xj, pos_i, pos_j, ew, ebias, aw, abias, k):
    e, c1 = xj.shape
    nq = e // k
    d = aw.shape[0]
    qb = min(64 if d >= 256 else 128, nq)
    eb = qb * k
    return pl.pallas_call(
        functools.partial(_lfa_body, k=k, qb=qb),
        grid=(nq // qb,),
        in_specs=[
            pl.BlockSpec((eb, c1), lambda i: (i, 0)),
            pl.BlockSpec((eb, 3), lambda i: (i, 0)),
            pl.BlockSpec((eb, 3), lambda i: (i, 0)),
            pl.BlockSpec(ew.shape, lambda i: (0, 0)),
            pl.BlockSpec((1, ew.shape[0]), lambda i: (0, 0)),
            pl.BlockSpec(aw.shape, lambda i: (0, 0)),
            pl.BlockSpec((1, d), lambda i: (0, 0)),
        ],
        out_specs=pl.BlockSpec((qb, d), lambda i: (i, 0)),
        out_shape=jax.ShapeDtypeStruct((nq, d), jnp.float32),
    )(xj, pos_i, pos_j, ew, ebias.reshape(1, -1), aw, abias.reshape(1, -1))


def _sc_gather(table, idx):
    v, dd = table.shape
    b = idx.shape[0]
    info = plsc.get_sparse_core_info()
    nw = info.num_cores * info.num_subcores
    bpw = b // nw
    ch = min(128, bpw)
    nch = bpw // ch
    mesh = plsc.VectorSubcoreMesh(core_axis_name="c", subcore_axis_name="s")

    @functools.partial(
        pl.kernel, mesh=mesh,
        compiler_params=pltpu.CompilerParams(use_tc_tiling_on_sc=False),
        out_type=jax.ShapeDtypeStruct((b, dd), jnp.float32),
        scratch_types=[
            pltpu.VMEM((bpw,), jnp.int32),
            pltpu.VMEM((bpw, dd), jnp.float32),
            pltpu.SemaphoreType.DMA,
        ],
    )
    def gk(table_hbm, idx_hbm, out_hbm, idx_v, rows_v, sem):
        wid = lax.axis_index("s") * info.num_cores + lax.axis_index("c")
        base = wid * bpw
        pltpu.sync_copy(idx_hbm.at[pl.ds(base, bpw)], idx_v)

        def body(j, carry):
            off = pl.multiple_of(j * ch, 8)
            pltpu.async_copy(table_hbm.at[idx_v.at[pl.ds(off, ch)]],
                             rows_v.at[pl.ds(off, ch)], sem).wait()
            return carry

        lax.fori_loop(0, nch, body, 0)
        pltpu.sync_copy(rows_v, out_hbm.at[pl.ds(base, bpw)])

    return gk(table, idx)


def _pad_cols(a, mult):
    n, c = a.shape
    pad = (-c) % mult
    if pad:
        a = jnp.concatenate([a, jnp.zeros((n, pad), a.dtype)], axis=1)
    return a


def _gather_rows(table, idx, want_cols):
    tp = _pad_cols(table, 16)
    return _sc_gather(tp, idx)[:, :want_cols]


def _block_fwd(params, pfx, x, pos, dec, k):
    n = x.shape[0]
    nq = n // dec
    q = pos[::dec]
    nbrs = _knn(q.T, pos, k)
    col = nbrs.T.reshape(-1)
    sc = _linear(x, params[pfx + "_sc_W"], params[pfx + "_sc_b"], "lrelu", 0.2)
    hm = _linear(x, params[pfx + "_m1_W"], params[pfx + "_m1_b"], "lrelu", 0.2)
    pos_j = _gather_rows(pos, col, 3)
    pos_i = jnp.repeat(pos[:nq], k, axis=0)
    xj = _gather_rows(hm, col, hm.shape[1])
    h1q = _lfa(xj, pos_i, pos_j,
               params[pfx + "_l1_e_W"], params[pfx + "_l1_e_b"],
               params[pfx + "_l1_a_W"], params[pfx + "_l1_a_b"], k)
    d1 = h1q.shape[1]
    h1full = jnp.concatenate(
        [h1q, jnp.zeros((n - nq, d1), jnp.float32)], axis=0)
    xj2 = _gather_rows(h1full, col, d1)
    h2q = _lfa(xj2, pos_i, pos_j,
               params[pfx + "_l2_e_W"], params[pfx + "_l2_e_b"],
               params[pfx + "_l2_a_W"], params[pfx + "_l2_a_b"], k)
    d2 = h2q.shape[1]
    h2full = jnp.concatenate(
        [h2q, jnp.zeros((n - nq, d2), jnp.float32)], axis=0)
    h4 = _linear_res(h2full, params[pfx + "_m2_W"], params[pfx + "_m2_b"], sc)
    return h4[::dec], q


def _fp_fwd(params, pfx, xh, pos, pos_skip, x_skip):
    nn = _nn1(pos_skip.T, pos)
    xi = _gather_rows(xh, nn, xh.shape[1])
    if x_skip is not None:
        xi = jnp.concatenate([xi, x_skip], axis=1)
    return _linear(xi, params[pfx + "_W"], params[pfx + "_b"], "none")


def _head_body(x_ref, w1_ref, b1_ref, w2_ref, b2_ref, w3_ref, b3_ref, o_ref):
    h = lax.dot_general(x_ref[...], w1_ref[...], (((1,), (1,)), ((), ())),
                        preferred_element_type=jnp.float32) + b1_ref[...]
    h = jnp.maximum(h, 0.0)
    h = lax.dot_general(h, w2_ref[...], (((1,), (1,)), ((), ())),
                        preferred_element_type=jnp.float32) + b2_ref[...]
    h = lax.dot_general(h, w3_ref[...], (((1,), (1,)), ((), ())),
                        preferred_element_type=jnp.float32) + b3_ref[...]
    o_ref[...] = h


def _head(x, w1, b1, w2, b2, w3, b3):
    n = x.shape[0]
    nb = min(512, n)
    d3 = w3.shape[0]
    return pl.pallas_call(
        _head_body,
        grid=(n // nb,),
        in_specs=[
            pl.BlockSpec((nb, x.shape[1]), lambda i: (i, 0)),
            pl.BlockSpec(w1.shape, lambda i: (0, 0)),
            pl.BlockSpec((1, w1.shape[0]), lambda i: (0, 0)),
            pl.BlockSpec(w2.shape, lambda i: (0, 0)),
            pl.BlockSpec((1, w2.shape[0]), lambda i: (0, 0)),
            pl.BlockSpec(w3.shape, lambda i: (0, 0)),
            pl.BlockSpec((1, d3), lambda i: (0, 0)),
        ],
        out_specs=pl.BlockSpec((nb, d3), lambda i: (i, 0)),
        out_shape=jax.ShapeDtypeStruct((n, d3), jnp.float32),
    )(x, w1, b1.reshape(1, -1), w2, b2.reshape(1, -1), w3, b3.reshape(1, -1))


def kernel(x, pos, batch, params):
    x0, p0 = x, pos
    x1, p1 = _block_fwd(params, "b1", x0, p0, 4, _K)
    x2, p2 = _block_fwd(params, "b2", x1, p1, 4, _K)
    x3, p3 = _block_fwd(params, "b3", x2, p2, 4, _K)
    x4, p4 = _block_fwd(params, "b4", x3, p3, 4, _K)
    h = _linear(x4, params["mlp1_W"], params["mlp1_b"], "none")
    h = _fp_fwd(params, "fp4", h, p4, p3, x3)
    h = _fp_fwd(params, "fp3", h, p3, p2, x2)
    h = _fp_fwd(params, "fp2", h, p2, p1, x1)
    h = _fp_fwd(params, "fp1", h, p1, p0, x0)
    lw = jnp.concatenate(
        [params["lin_W"], jnp.zeros((3, params["lin_W"].shape[1]))], axis=0)
    lb = jnp.concatenate([params["lin_b"], jnp.zeros((3,))], axis=0)
    out = _head(h, params["h1_W"], params["h1_b"],
                params["h2_W"], params["h2_b"], lw, lb)
    return out[:, :13]


# speedup vs baseline: 3.2921x; 1.1843x over previous
import functools

import jax
import jax.numpy as jnp
from jax import lax
from jax.experimental import pallas as pl
from jax.experimental.pallas import tpu as pltpu
from jax.experimental.pallas import tpu_sc as plsc

_K = 16


def _lrelu(v, s):
    return jnp.where(v >= 0, v, s * v)


def _g_prep_body(p_ref, g_ref):
    p = p_ref[...]
    g_ref[...] = jnp.concatenate(
        [-2.0 * p, jnp.sum(p * p, axis=1, keepdims=True)], axis=1)


def _g_prep(pos):
    n = pos.shape[0]
    nb = min(8192, n)
    return pl.pallas_call(
        _g_prep_body,
        grid=(n // nb,),
        in_specs=[pl.BlockSpec((nb, 3), lambda i: (i, 0))],
        out_specs=pl.BlockSpec((nb, 4), lambda i: (i, 0)),
        out_shape=jax.ShapeDtypeStruct((n, 4), jnp.float32),
    )(pos)


def _knn_body(q_ref, g_ref, o_ref, *, n, st, k):
    qb = q_ref.shape[1]
    qh = jnp.concatenate(
        [q_ref[...], jnp.ones((1, qb), jnp.float32)], axis=0)
    it = lax.broadcasted_iota(jnp.int32, (st, qb), 0)
    slot = lax.broadcasted_iota(jnp.int32, (k, 1), 0)

    def tile_step(t, carry):
        rv, ri = carry
        gt = g_ref[pl.ds(t * st, st), :]
        d = lax.dot_general(gt, qh, (((1,), (0,)), ((), ())),
                            preferred_element_type=jnp.float32)

        w0 = jnp.max(rv, axis=0, keepdims=True)
        m0 = jnp.min(d, axis=0, keepdims=True)
        go0 = jnp.any(m0 < w0)

        def cond(c):
            return c[0]

        def round_step(c):
            _, d, m, rv, ri, w = c
            a = jnp.where(d == m, it, n)
            i = jnp.min(a, axis=0, keepdims=True)
            d = jnp.where(it == i, jnp.inf, d)
            ws = jnp.where(rv == w, slot, k)
            s0 = jnp.min(ws, axis=0, keepdims=True)
            sel = (slot == s0) & (m < w)
            rv = jnp.where(sel, m, rv)
            ri = jnp.where(sel, i + t * st, ri)
            w = jnp.max(rv, axis=0, keepdims=True)
            m = jnp.min(d, axis=0, keepdims=True)
            return jnp.any(m < w), d, m, rv, ri, w

        _, _, _, rv, ri, _ = lax.while_loop(
            cond, round_step, (go0, d, m0, rv, ri, w0))
        return rv, ri

    rv0 = jnp.full((k, qb), jnp.inf, jnp.float32)
    ri0 = jnp.zeros((k, qb), jnp.int32)
    _, ri = lax.fori_loop(0, n // st, tile_step, (rv0, ri0))
    o_ref[...] = ri


def _knn(q_t, g, k):
    nq = q_t.shape[1]
    n = g.shape[0]
    qb = min(128, nq)
    st = min(1024, n)
    return pl.pallas_call(
        functools.partial(_knn_body, n=n, st=st, k=k),
        grid=(nq // qb,),
        in_specs=[
            pl.BlockSpec((3, qb), lambda i: (0, i)),
            pl.BlockSpec((n, 4), lambda i: (0, 0)),
        ],
        out_specs=pl.BlockSpec((k, qb), lambda i: (0, i)),
        out_shape=jax.ShapeDtypeStruct((k, nq), jnp.int32),
    )(q_t, g)


def _nn1_body(q_ref, g_ref, o_ref, *, n, st):
    qb = q_ref.shape[1]
    qh = jnp.concatenate(
        [q_ref[...], jnp.ones((1, qb), jnp.float32)], axis=0)
    it = lax.broadcasted_iota(jnp.int32, (st, qb), 0)

    def tile_step(t, carry):
        bv, bi = carry
        gt = g_ref[pl.ds(t * st, st), :]
        d = lax.dot_general(gt, qh, (((1,), (0,)), ((), ())),
                            preferred_element_type=jnp.float32)
        m = jnp.min(d, axis=0, keepdims=True)
        a = jnp.where(d == m, it, n)
        i = jnp.min(a, axis=0, keepdims=True) + t * st
        better = m < bv
        bv = jnp.where(better, m, bv)
        bi = jnp.where(better, i, bi)
        return bv, bi

    bv0 = jnp.full((1, qb), jnp.inf, jnp.float32)
    bi0 = jnp.zeros((1, qb), jnp.int32)
    _, bi = lax.fori_loop(0, n // st, tile_step, (bv0, bi0))
    o_ref[...] = bi


def _nn1(q_t, g):
    nq = q_t.shape[1]
    n = g.shape[0]
    qb = min(128, nq)
    st = min(2048, n)
    out = pl.pallas_call(
        functools.partial(_nn1_body, n=n, st=st),
        grid=(nq // qb,),
        in_specs=[
            pl.BlockSpec((3, qb), lambda i: (0, i)),
            pl.BlockSpec((n, 4), lambda i: (0, 0)),
        ],
        out_specs=pl.BlockSpec((1, qb), lambda i: (0, i)),
        out_shape=jax.ShapeDtypeStruct((1, nq), jnp.int32),
    )(q_t, g)
    return out[0]


def _lin_body(x_ref, w_ref, b_ref, o_ref, *, act, slope):
    y = lax.dot_general(x_ref[...], w_ref[...], (((1,), (1,)), ((), ())),
                        preferred_element_type=jnp.float32)
    y = y + b_ref[...]
    if act == "lrelu":
        y = _lrelu(y, slope)
    elif act == "relu":
        y = jnp.maximum(y, 0.0)
    o_ref[...] = y


def _lin_res_body(x_ref, w_ref, b_ref, r_ref, o_ref, *, slope, slope2):
    y = lax.dot_general(x_ref[...], w_ref[...], (((1,), (1,)), ((), ())),
                        preferred_element_type=jnp.float32)
    y = _lrelu(y + b_ref[...], slope)
    o_ref[...] = _lrelu(y + r_ref[...], slope2)


def _linear(x, w, b, act="none", slope=0.2):
    n, din = x.shape
    dout = w.shape[0]
    nb = min(512, n)
    return pl.pallas_call(
        functools.partial(_lin_body, act=act, slope=slope),
        grid=(n // nb,),
        in_specs=[
            pl.BlockSpec((nb, din), lambda i: (i, 0)),
            pl.BlockSpec((dout, din), lambda i: (0, 0)),
            pl.BlockSpec((1, dout), lambda i: (0, 0)),
        ],
        out_specs=pl.BlockSpec((nb, dout), lambda i: (i, 0)),
        out_shape=jax.ShapeDtypeStruct((n, dout), jnp.float32),
    )(x, w, b.reshape(1, -1))


def _linear_res(x, w, b, res, slope=0.2, slope2=0.01):
    n, din = x.shape
    dout = w.shape[0]
    nb = min(512, n)
    return pl.pallas_call(
        functools.partial(_lin_res_body, slope=slope, slope2=slope2),
        grid=(n // nb,),
        in_specs=[
            pl.BlockSpec((nb, din), lambda i: (i, 0)),
            pl.BlockSpec((dout, din), lambda i: (0, 0)),
            pl.BlockSpec((1, dout), lambda i: (0, 0)),
            pl.BlockSpec((nb, dout), lambda i: (i, 0)),
        ],
        out_specs=pl.BlockSpec((nb, dout), lambda i: (i, 0)),
        out_shape=jax.ShapeDtypeStruct((n, dout), jnp.float32),
    )(x, w, b.reshape(1, -1), res)


def _lfa_body(xj_ref, pi_ref, pj_ref, ew_ref, eb_ref, aw_ref, ab_ref, o_ref,
              *, k, qb):
    pi = pi_ref[...]
    pj = pj_ref[...]
    dist = pj - pi
    eu = jnp.sum(jnp.abs(dist), axis=1, keepdims=True)
    rel = jnp.concatenate([pi, pj, dist, eu], axis=1)
    lse = lax.dot_general(rel, ew_ref[...], (((1,), (1,)), ((), ())),
                          preferred_element_type=jnp.float32) + eb_ref[...]
    out1 = jnp.concatenate([xj_ref[...], lse], axis=1)
    logits = lax.dot_general(out1, aw_ref[...], (((1,), (1,)), ((), ())),
                             preferred_element_type=jnp.float32) + ab_ref[...]
    z = logits - jnp.max(logits, axis=1, keepdims=True)
    e = jnp.exp(z)
    att = e / jnp.sum(e, axis=1, keepdims=True)
    msg = att * out1
    eb = qb * k
    rows = lax.broadcasted_iota(jnp.int32, (qb, eb), 0)
    cols = lax.broadcasted_iota(jnp.int32, (qb, eb), 1)
    sel = jnp.where(cols // k == rows, 1.0, 0.0)
    o_ref[...] = lax.dot_general(sel, msg, (((1,), (0,)), ((), ())),
                                 preferred_element_type=jnp.float32)


def _lfa(xj, pos_i, pos_j, ew, ebias, aw, abias, k):
    e, c1 = xj.shape
    nq = e // k
    d = aw.shape[0]
    qb = min(64 if d >= 256 else 128, nq)
    eb = qb * k
    return pl.pallas_call(
        functools.partial(_lfa_body, k=k, qb=qb),
        grid=(nq // qb,),
        in_specs=[
            pl.BlockSpec((eb, c1), lambda i: (i, 0)),
            pl.BlockSpec((eb, 3), lambda i: (i, 0)),
            pl.BlockSpec((eb, 3), lambda i: (i, 0)),
            pl.BlockSpec(ew.shape, lambda i: (0, 0)),
            pl.BlockSpec((1, ew.shape[0]), lambda i: (0, 0)),
            pl.BlockSpec(aw.shape, lambda i: (0, 0)),
            pl.BlockSpec((1, d), lambda i: (0, 0)),
        ],
        out_specs=pl.BlockSpec((qb, d), lambda i: (i, 0)),
        out_shape=jax.ShapeDtypeStruct((nq, d), jnp.float32),
    )(xj, pos_i, pos_j, ew, ebias.reshape(1, -1), aw, abias.reshape(1, -1))


def _sc_gather(table, idx):
    v, dd = table.shape
    b = idx.shape[0]
    info = plsc.get_sparse_core_info()
    nw = info.num_cores * info.num_subcores
    bpw = b // nw
    ch = min(128, bpw)
    nch = bpw // ch
    mesh = plsc.VectorSubcoreMesh(core_axis_name="c", subcore_axis_name="s")

    @functools.partial(
        pl.kernel, mesh=mesh,
        compiler_params=pltpu.CompilerParams(use_tc_tiling_on_sc=False),
        out_type=jax.ShapeDtypeStruct((b, dd), jnp.float32),
        scratch_types=[
            pltpu.VMEM((bpw,), jnp.int32),
            pltpu.VMEM((bpw, dd), jnp.float32),
            pltpu.SemaphoreType.DMA,
        ],
    )
    def gk(table_hbm, idx_hbm, out_hbm, idx_v, rows_v, sem):
        wid = lax.axis_index("s") * info.num_cores + lax.axis_index("c")
        base = wid * bpw
        pltpu.sync_copy(idx_hbm.at[pl.ds(base, bpw)], idx_v)

        def body(j, carry):
            off = pl.multiple_of(j * ch, 8)
            pltpu.async_copy(table_hbm.at[idx_v.at[pl.ds(off, ch)]],
                             rows_v.at[pl.ds(off, ch)], sem).wait()
            return carry

        lax.fori_loop(0, nch, body, 0)
        pltpu.sync_copy(rows_v, out_hbm.at[pl.ds(base, bpw)])

    return gk(table, idx)


def _pad_cols(a, mult):
    n, c = a.shape
    pad = (-c) % mult
    if pad:
        a = jnp.concatenate([a, jnp.zeros((n, pad), a.dtype)], axis=1)
    return a


def _gather_rows(table, idx, want_cols):
    tp = _pad_cols(table, 16)
    return _sc_gather(tp, idx)[:, :want_cols]


def _block_fwd(params, pfx, x, pos, g, dec, k):
    n = x.shape[0]
    nq = n // dec
    q = pos[::dec]
    nbrs = _knn(q.T, g, k)
    col = nbrs.T.reshape(-1)
    sc = _linear(x, params[pfx + "_sc_W"], params[pfx + "_sc_b"], "lrelu", 0.2)
    hm = _linear(x, params[pfx + "_m1_W"], params[pfx + "_m1_b"], "lrelu", 0.2)
    pos_j = _gather_rows(pos, col, 3)
    pos_i = jnp.repeat(pos[:nq], k, axis=0)
    xj = _gather_rows(hm, col, hm.shape[1])
    h1q = _lfa(xj, pos_i, pos_j,
               params[pfx + "_l1_e_W"], params[pfx + "_l1_e_b"],
               params[pfx + "_l1_a_W"], params[pfx + "_l1_a_b"], k)
    d1 = h1q.shape[1]
    h1full = jnp.concatenate(
        [h1q, jnp.zeros((n - nq, d1), jnp.float32)], axis=0)
    xj2 = _gather_rows(h1full, col, d1)
    h2q = _lfa(xj2, pos_i, pos_j,
               params[pfx + "_l2_e_W"], params[pfx + "_l2_e_b"],
               params[pfx + "_l2_a_W"], params[pfx + "_l2_a_b"], k)
    d2 = h2q.shape[1]
    h2full = jnp.concatenate(
        [h2q, jnp.zeros((n - nq, d2), jnp.float32)], axis=0)
    h4 = _linear_res(h2full, params[pfx + "_m2_W"], params[pfx + "_m2_b"], sc)
    return h4[::dec], q


def _fp_fwd(params, pfx, xh, g, pos_skip, x_skip):
    nn = _nn1(pos_skip.T, g)
    xi = _gather_rows(xh, nn, xh.shape[1])
    if x_skip is not None:
        xi = jnp.concatenate([xi, x_skip], axis=1)
    return _linear(xi, params[pfx + "_W"], params[pfx + "_b"], "none")


def _head_body(x_ref, w1_ref, b1_ref, w2_ref, b2_ref, w3_ref, b3_ref, o_ref):
    h = lax.dot_general(x_ref[...], w1_ref[...], (((1,), (1,)), ((), ())),
                        preferred_element_type=jnp.float32) + b1_ref[...]
    h = jnp.maximum(h, 0.0)
    h = lax.dot_general(h, w2_ref[...], (((1,), (1,)), ((), ())),
                        preferred_element_type=jnp.float32) + b2_ref[...]
    h = lax.dot_general(h, w3_ref[...], (((1,), (1,)), ((), ())),
                        preferred_element_type=jnp.float32) + b3_ref[...]
    o_ref[...] = h


def _head(x, w1, b1, w2, b2, w3, b3):
    n = x.shape[0]
    nb = min(512, n)
    d3 = w3.shape[0]
    return pl.pallas_call(
        _head_body,
        grid=(n // nb,),
        in_specs=[
            pl.BlockSpec((nb, x.shape[1]), lambda i: (i, 0)),
            pl.BlockSpec(w1.shape, lambda i: (0, 0)),
            pl.BlockSpec((1, w1.shape[0]), lambda i: (0, 0)),
            pl.BlockSpec(w2.shape, lambda i: (0, 0)),
            pl.BlockSpec((1, w2.shape[0]), lambda i: (0, 0)),
            pl.BlockSpec(w3.shape, lambda i: (0, 0)),
            pl.BlockSpec((1, d3), lambda i: (0, 0)),
        ],
        out_specs=pl.BlockSpec((nb, d3), lambda i: (i, 0)),
        out_shape=jax.ShapeDtypeStruct((n, d3), jnp.float32),
    )(x, w1, b1.reshape(1, -1), w2, b2.reshape(1, -1), w3, b3.reshape(1, -1))


def kernel(x, pos, batch, params):
    x0, p0 = x, pos
    g0 = _g_prep(p0)
    x1, p1 = _block_fwd(params, "b1", x0, p0, g0, 4, _K)
    g1 = _g_prep(p1)
    x2, p2 = _block_fwd(params, "b2", x1, p1, g1, 4, _K)
    g2 = _g_prep(p2)
    x3, p3 = _block_fwd(params, "b3", x2, p2, g2, 4, _K)
    g3 = _g_prep(p3)
    x4, p4 = _block_fwd(params, "b4", x3, p3, g3, 4, _K)
    g4 = _g_prep(p4)
    h = _linear(x4, params["mlp1_W"], params["mlp1_b"], "none")
    h = _fp_fwd(params, "fp4", h, g4, p3, x3)
    h = _fp_fwd(params, "fp3", h, g3, p2, x2)
    h = _fp_fwd(params, "fp2", h, g2, p1, x1)
    h = _fp_fwd(params, "fp1", h, g1, p0, x0)
    lw = jnp.concatenate(
        [params["lin_W"], jnp.zeros((3, params["lin_W"].shape[1]))], axis=0)
    lb = jnp.concatenate([params["lin_b"], jnp.zeros((3,))], axis=0)
    out = _head(h, params["h1_W"], params["h1_b"],
                params["h2_W"], params["h2_b"], lw, lb)
    return out[:, :13]
